# Initial kernel scaffold; baseline (speedup 1.0000x reference)
import jax
import jax.numpy as jnp
from jax.experimental import pallas as pl


def kernel(x, edge_index, edge_probs, time_decay, node_bias, edge_weight):
    raise NotImplementedError("write your pallas kernel here")


# speedup vs baseline: 111.5658x; 111.5658x over previous
import jax
import jax.numpy as jnp
from jax import lax
from jax.experimental import pallas as pl
from jax.experimental.pallas import tpu as pltpu
from jax.experimental.pallas import tpu_sc as plsc

NSTEPS = 8
N = 100000
E = 6400000
NC, NS, L = 2, 16, 16
NPAD = 102400
NPT = NPAD // NS
EPT = E // NS
CHUNK = 8000
NCHUNKS = EPT // CHUNK


def _body(x_hbm, src_hbm, dst_hbm, ep_hbm, coefs_hbm, bias_hbm, out_hbm,
          cur_s, acc_s,
          xbuf, abuf, zbuf, sbuf, src_v, dst_v, prob_v, val_v, cvec, bvec):
    cid = lax.axis_index("c")
    sid = lax.axis_index("s")
    nb = sid * NPT

    pltpu.sync_copy(bias_hbm, bvec)
    pltpu.sync_copy(x_hbm.at[pl.ds(nb, NPT)], xbuf)

    def init_body(i, c):
        sl = pl.ds(i * L, L)
        sbuf[sl] = 1.0 - xbuf[sl]
        zbuf[sl] = jnp.zeros((L,), jnp.float32)
        return c
    lax.fori_loop(0, NPT // L, init_body, 0)

    pltpu.sync_copy(xbuf, cur_s.at[pl.ds(nb, NPT)])
    pltpu.sync_copy(zbuf, acc_s.at[pl.ds(nb, NPT)])
    plsc.subcore_barrier()

    for k in range(NSTEPS):
        pltpu.sync_copy(coefs_hbm.at[k], cvec)

        def chunk_body(j, c):
            base = sid * EPT + j * CHUNK
            pltpu.sync_copy(src_hbm.at[pl.ds(base, CHUNK)], src_v)
            pltpu.sync_copy(dst_hbm.at[pl.ds(base, CHUNK)], dst_v)
            pltpu.sync_copy(ep_hbm.at[pl.ds(base, CHUNK)], prob_v)
            pltpu.sync_copy(cur_s.at[src_v], val_v)
            def mul_body(i, c2):
                sl = pl.ds(i * L, L)
                val_v[sl] = val_v[sl] * prob_v[sl] * cvec[...]
                return c2
            lax.fori_loop(0, CHUNK // L, mul_body, 0)
            pltpu.sync_copy(val_v, acc_s.at[dst_v], add=True)
            return c
        lax.fori_loop(0, NCHUNKS, chunk_body, 0)
        plsc.subcore_barrier()

        pltpu.sync_copy(acc_s.at[pl.ds(nb, NPT)], abuf)
        def node_body(i, c):
            sl = pl.ds(i * L, L)
            cur = abuf[sl] + bvec[...]
            abuf[sl] = cur
            sbuf[sl] = sbuf[sl] * (1.0 - cur)
            return c
        lax.fori_loop(0, NPT // L, node_body, 0)
        pltpu.sync_copy(abuf, cur_s.at[pl.ds(nb, NPT)])
        pltpu.sync_copy(zbuf, acc_s.at[pl.ds(nb, NPT)])
        plsc.subcore_barrier()

    @pl.when(cid == 0)
    def _():
        def out_body(i, c):
            sl = pl.ds(i * L, L)
            v = 1.0 - sbuf[sl]
            sbuf[sl] = jnp.minimum(jnp.maximum(v, 0.0), 1.0)
            return c
        lax.fori_loop(0, NPT // L, out_body, 0)
        pltpu.sync_copy(sbuf, out_hbm.at[pl.ds(nb, NPT)])


def kernel(x, edge_index, edge_probs, time_decay, node_bias, edge_weight):
    x_pad = jnp.pad(x[:, 0], (0, NPAD - N))
    ei = edge_index.astype(jnp.int32)
    coefs = edge_weight.astype(jnp.float32) * jnp.exp(
        -(time_decay.astype(jnp.float32) ** 2))
    coefs16 = jnp.broadcast_to(coefs[:, None], (NSTEPS, L))
    bias16 = jnp.full((L,), node_bias, jnp.float32)

    mesh = plsc.VectorSubcoreMesh(core_axis_name="c", subcore_axis_name="s",
                                  num_cores=NC, num_subcores=NS)
    run = pl.kernel(
        _body,
        out_type=jax.ShapeDtypeStruct((NPAD,), jnp.float32),
        mesh=mesh,
        scratch_types=[
            pltpu.VMEM_SHARED((NPAD,), jnp.float32),
            pltpu.VMEM_SHARED((NPAD,), jnp.float32),
            pltpu.VMEM((NPT,), jnp.float32),
            pltpu.VMEM((NPT,), jnp.float32),
            pltpu.VMEM((NPT,), jnp.float32),
            pltpu.VMEM((NPT,), jnp.float32),
            pltpu.VMEM((CHUNK,), jnp.int32),
            pltpu.VMEM((CHUNK,), jnp.int32),
            pltpu.VMEM((CHUNK,), jnp.float32),
            pltpu.VMEM((CHUNK,), jnp.float32),
            pltpu.VMEM((L,), jnp.float32),
            pltpu.VMEM((L,), jnp.float32),
        ],
    )
    out = run(x_pad, ei[0], ei[1], edge_probs.astype(jnp.float32),
              coefs16, bias16)
    return out[:N, None]


# speedup vs baseline: 237.7516x; 2.1310x over previous
import jax
import jax.numpy as jnp
from jax import lax
from jax.experimental import pallas as pl
from jax.experimental.pallas import tpu as pltpu
from jax.experimental.pallas import tpu_sc as plsc

NSTEPS = 8
N = 100000
E = 6400000
NC, NS, L = 2, 16, 16
NPAD = 102400
NPT = NPAD // NS
NPW = NPAD // (NC * NS)
EPSC = E // NC
EPT = EPSC // NS
CHUNK = 8000
NCHUNKS = EPT // CHUNK

_f32 = jnp.float32


def _edge_pass(src_hbm, dst_hbm, ep_hbm, cur_s, acc_s,
               src_v, dst_v, prob_v, val_v, cid, sid):
    def chunk_body(j, c):
        base = cid * EPSC + sid * EPT + j * CHUNK
        pltpu.sync_copy(src_hbm.at[pl.ds(base, CHUNK)], src_v)
        pltpu.sync_copy(dst_hbm.at[pl.ds(base, CHUNK)], dst_v)
        pltpu.sync_copy(ep_hbm.at[pl.ds(base, CHUNK)], prob_v)
        pltpu.sync_copy(cur_s.at[src_v], val_v)
        def mul_body(i, c2):
            sl = pl.ds(i * L, L)
            val_v[sl] = val_v[sl] * prob_v[sl]
            return c2
        lax.fori_loop(0, CHUNK // L, mul_body, 0)
        pltpu.sync_copy(val_v, acc_s.at[dst_v], add=True)
        return c
    lax.fori_loop(0, NCHUNKS, chunk_body, 0)


def _write_partial(acc_s, p0_out, p1_out, cid, nb):
    sl = pl.ds(nb, NPT)
    @pl.when(cid == 0)
    def _():
        pltpu.sync_copy(acc_s.at[sl], p0_out.at[sl])
    @pl.when(cid == 1)
    def _():
        pltpu.sync_copy(acc_s.at[sl], p1_out.at[sl])


def _body_first(x_hbm, src_hbm, dst_hbm, ep_hbm, coef_hbm,
                p0_out, p1_out, surv_out,
                cur_s, acc_s, xbuf, abuf, zbuf, src_v, dst_v, prob_v, val_v,
                cvec):
    cid = lax.axis_index("c")
    sid = lax.axis_index("s")
    nb = sid * NPT

    pltpu.sync_copy(coef_hbm, cvec)
    pltpu.sync_copy(x_hbm.at[pl.ds(nb, NPT)], xbuf)

    def init_body(i, c):
        sl = pl.ds(i * L, L)
        abuf[sl] = cvec[...] * xbuf[sl]
        zbuf[sl] = jnp.zeros((L,), _f32)
        return c
    lax.fori_loop(0, NPT // L, init_body, 0)

    pltpu.sync_copy(abuf, cur_s.at[pl.ds(nb, NPT)])
    pltpu.sync_copy(zbuf, acc_s.at[pl.ds(nb, NPT)])

    @pl.when(cid == 0)
    def _():
        def sbody(i, c):
            sl = pl.ds(i * L, L)
            xbuf[sl] = 1.0 - xbuf[sl]
            return c
        lax.fori_loop(0, NPT // L, sbody, 0)
        pltpu.sync_copy(xbuf, surv_out.at[pl.ds(nb, NPT)])

    plsc.subcore_barrier()
    _edge_pass(src_hbm, dst_hbm, ep_hbm, cur_s, acc_s,
               src_v, dst_v, prob_v, val_v, cid, sid)
    plsc.subcore_barrier()
    _write_partial(acc_s, p0_out, p1_out, cid, nb)


def _body_mid(p0_hbm, p1_hbm, surv_hbm, src_hbm, dst_hbm, ep_hbm,
              coef_hbm, bias_hbm,
              p0_out, p1_out, surv_out,
              cur_s, acc_s, p0buf, p1buf, sbuf, zbuf,
              src_v, dst_v, prob_v, val_v, cvec, bvec):
    cid = lax.axis_index("c")
    sid = lax.axis_index("s")
    nb = sid * NPT

    pltpu.sync_copy(coef_hbm, cvec)
    pltpu.sync_copy(bias_hbm, bvec)
    pltpu.sync_copy(p0_hbm.at[pl.ds(nb, NPT)], p0buf)
    pltpu.sync_copy(p1_hbm.at[pl.ds(nb, NPT)], p1buf)

    def comb_body(i, c):
        sl = pl.ds(i * L, L)
        cur = p0buf[sl] + p1buf[sl] + bvec[...]
        p0buf[sl] = cur * cvec[...]
        p1buf[sl] = 1.0 - cur
        zbuf[sl] = jnp.zeros((L,), _f32)
        return c
    lax.fori_loop(0, NPT // L, comb_body, 0)

    pltpu.sync_copy(p0buf, cur_s.at[pl.ds(nb, NPT)])
    pltpu.sync_copy(zbuf, acc_s.at[pl.ds(nb, NPT)])

    @pl.when(cid == 0)
    def _():
        pltpu.sync_copy(surv_hbm.at[pl.ds(nb, NPT)], sbuf)
        def sbody(i, c):
            sl = pl.ds(i * L, L)
            sbuf[sl] = sbuf[sl] * p1buf[sl]
            return c
        lax.fori_loop(0, NPT // L, sbody, 0)
        pltpu.sync_copy(sbuf, surv_out.at[pl.ds(nb, NPT)])

    plsc.subcore_barrier()
    _edge_pass(src_hbm, dst_hbm, ep_hbm, cur_s, acc_s,
               src_v, dst_v, prob_v, val_v, cid, sid)
    plsc.subcore_barrier()
    _write_partial(acc_s, p0_out, p1_out, cid, nb)


def _body_fin(p0_hbm, p1_hbm, surv_hbm, bias_hbm, out_hbm,
              p0buf, p1buf, sbuf, bvec):
    cid = lax.axis_index("c")
    sid = lax.axis_index("s")
    nb = (cid * NS + sid) * NPW

    pltpu.sync_copy(bias_hbm, bvec)
    pltpu.sync_copy(p0_hbm.at[pl.ds(nb, NPW)], p0buf)
    pltpu.sync_copy(p1_hbm.at[pl.ds(nb, NPW)], p1buf)
    pltpu.sync_copy(surv_hbm.at[pl.ds(nb, NPW)], sbuf)

    def fin_body(i, c):
        sl = pl.ds(i * L, L)
        cur = p0buf[sl] + p1buf[sl] + bvec[...]
        v = 1.0 - sbuf[sl] * (1.0 - cur)
        p0buf[sl] = jnp.minimum(jnp.maximum(v, 0.0), 1.0)
        return c
    lax.fori_loop(0, NPW // L, fin_body, 0)

    pltpu.sync_copy(p0buf, out_hbm.at[pl.ds(nb, NPW)])


def kernel(x, edge_index, edge_probs, time_decay, node_bias, edge_weight):
    x_pad = jnp.pad(x[:, 0], (0, NPAD - N))
    ei = edge_index.astype(jnp.int32)
    ep = edge_probs.astype(_f32)
    coefs = edge_weight.astype(_f32) * jnp.exp(-(time_decay.astype(_f32) ** 2))
    coefs16 = jnp.broadcast_to(coefs[:, None], (NSTEPS, L))
    bias16 = jnp.full((L,), node_bias, _f32)

    mesh = plsc.VectorSubcoreMesh(core_axis_name="c", subcore_axis_name="s",
                                  num_cores=NC, num_subcores=NS)
    node_arr = jax.ShapeDtypeStruct((NPAD,), _f32)

    edge_scratch = [
        pltpu.VMEM((CHUNK,), jnp.int32),
        pltpu.VMEM((CHUNK,), jnp.int32),
        pltpu.VMEM((CHUNK,), _f32),
        pltpu.VMEM((CHUNK,), _f32),
    ]
    spmem_scratch = [
        pltpu.VMEM_SHARED((NPAD,), _f32),
        pltpu.VMEM_SHARED((NPAD,), _f32),
    ]

    first = pl.kernel(
        _body_first,
        out_type=(node_arr, node_arr, node_arr),
        mesh=mesh,
        scratch_types=spmem_scratch + [
            pltpu.VMEM((NPT,), _f32),
            pltpu.VMEM((NPT,), _f32),
            pltpu.VMEM((NPT,), _f32),
        ] + edge_scratch + [
            pltpu.VMEM((L,), _f32),
        ],
    )
    mid = pl.kernel(
        _body_mid,
        out_type=(node_arr, node_arr, node_arr),
        mesh=mesh,
        scratch_types=spmem_scratch + [
            pltpu.VMEM((NPT,), _f32),
            pltpu.VMEM((NPT,), _f32),
            pltpu.VMEM((NPT,), _f32),
            pltpu.VMEM((NPT,), _f32),
        ] + edge_scratch + [
            pltpu.VMEM((L,), _f32),
            pltpu.VMEM((L,), _f32),
        ],
    )
    fin = pl.kernel(
        _body_fin,
        out_type=node_arr,
        mesh=mesh,
        scratch_types=[
            pltpu.VMEM((NPW,), _f32),
            pltpu.VMEM((NPW,), _f32),
            pltpu.VMEM((NPW,), _f32),
            pltpu.VMEM((L,), _f32),
        ],
    )

    p0, p1, surv = first(x_pad, ei[0], ei[1], ep, coefs16[0])
    for k in range(1, NSTEPS):
        p0, p1, surv = mid(p0, p1, surv, ei[0], ei[1], ep,
                           coefs16[k], bias16)
    out = fin(p0, p1, surv, bias16)
    return out[:N, None]


# speedup vs baseline: 330.8942x; 1.3918x over previous
import jax
import jax.numpy as jnp
from jax import lax
from jax.experimental import pallas as pl
from jax.experimental.pallas import tpu as pltpu
from jax.experimental.pallas import tpu_sc as plsc

NSTEPS = 8
N = 100000
E = 6400000
NC, NS, L = 2, 16, 16
NPAD = 102400
NPT = NPAD // NS
NPW = NPAD // (NC * NS)
EPSC = E // NC
EPT = EPSC // NS
CHUNK = 8000
NCHUNKS = EPT // CHUNK

_f32 = jnp.float32


def _edge_pass(src_hbm, dst_hbm, ep_hbm, cur_s, acc_s,
               srcs, dsts, probs, vals, lsems, gsem, ssems, cid, sid):
    ldesc, sdesc = {}, {}

    def issue_loads(c):
        b = c % 3
        base = cid * EPSC + sid * EPT + c * CHUNK
        ldesc[c] = (
            pltpu.async_copy(src_hbm.at[pl.ds(base, CHUNK)], srcs[b], lsems[b]),
            pltpu.async_copy(dst_hbm.at[pl.ds(base, CHUNK)], dsts[b], lsems[b]),
            pltpu.async_copy(ep_hbm.at[pl.ds(base, CHUNK)], probs[b], lsems[b]),
        )

    issue_loads(0)
    issue_loads(1)
    for c in range(NCHUNKS):
        b, vb = c % 3, c % 2
        for d in ldesc.pop(c):
            d.wait()
        if c >= 2:
            sdesc.pop(c - 2).wait()
        if c + 1 < NCHUNKS and c + 1 not in ldesc:
            issue_loads(c + 1)
        g = pltpu.async_copy(cur_s.at[srcs[b]], vals[vb], gsem)
        g.wait()
        def mul_body(i, c2, vb=vb, b=b):
            sl = pl.ds(i * L, L)
            vals[vb][sl] = vals[vb][sl] * probs[b][sl]
            return c2
        lax.fori_loop(0, CHUNK // L, mul_body, 0)
        sdesc[c] = pltpu.async_copy(vals[vb], acc_s.at[dsts[b]], ssems[vb],
                                    add=True)
    sdesc.pop(NCHUNKS - 2).wait()
    sdesc.pop(NCHUNKS - 1).wait()


def _write_partial(acc_s, p0_out, p1_out, cid, nb):
    sl = pl.ds(nb, NPT)
    @pl.when(cid == 0)
    def _():
        pltpu.sync_copy(acc_s.at[sl], p0_out.at[sl])
    @pl.when(cid == 1)
    def _():
        pltpu.sync_copy(acc_s.at[sl], p1_out.at[sl])


def _body_first(x_hbm, src_hbm, dst_hbm, ep_hbm, coef_hbm,
                p0_out, p1_out, surv_out,
                cur_s, acc_s, xbuf, abuf, zbuf,
                s0, s1, s2, d0, d1, d2, pr0, pr1, pr2, v0, v1,
                ls0, ls1, ls2, gsem, ss0, ss1,
                cvec):
    cid = lax.axis_index("c")
    sid = lax.axis_index("s")
    nb = sid * NPT

    pltpu.sync_copy(coef_hbm, cvec)
    pltpu.sync_copy(x_hbm.at[pl.ds(nb, NPT)], xbuf)

    def init_body(i, c):
        sl = pl.ds(i * L, L)
        abuf[sl] = cvec[...] * xbuf[sl]
        zbuf[sl] = jnp.zeros((L,), _f32)
        return c
    lax.fori_loop(0, NPT // L, init_body, 0)

    pltpu.sync_copy(abuf, cur_s.at[pl.ds(nb, NPT)])
    pltpu.sync_copy(zbuf, acc_s.at[pl.ds(nb, NPT)])

    @pl.when(cid == 0)
    def _():
        def sbody(i, c):
            sl = pl.ds(i * L, L)
            xbuf[sl] = 1.0 - xbuf[sl]
            return c
        lax.fori_loop(0, NPT // L, sbody, 0)
        pltpu.sync_copy(xbuf, surv_out.at[pl.ds(nb, NPT)])

    plsc.subcore_barrier()
    _edge_pass(src_hbm, dst_hbm, ep_hbm, cur_s, acc_s,
               (s0, s1, s2), (d0, d1, d2), (pr0, pr1, pr2), (v0, v1),
               (ls0, ls1, ls2), gsem, (ss0, ss1), cid, sid)
    plsc.subcore_barrier()
    _write_partial(acc_s, p0_out, p1_out, cid, nb)


def _body_mid(p0_hbm, p1_hbm, surv_hbm, src_hbm, dst_hbm, ep_hbm,
              coef_hbm, bias_hbm,
              p0_out, p1_out, surv_out,
              cur_s, acc_s, p0buf, p1buf, sbuf, zbuf,
              s0, s1, s2, d0, d1, d2, pr0, pr1, pr2, v0, v1,
              ls0, ls1, ls2, gsem, ss0, ss1,
              cvec, bvec):
    cid = lax.axis_index("c")
    sid = lax.axis_index("s")
    nb = sid * NPT

    pltpu.sync_copy(coef_hbm, cvec)
    pltpu.sync_copy(bias_hbm, bvec)
    pltpu.sync_copy(p0_hbm.at[pl.ds(nb, NPT)], p0buf)
    pltpu.sync_copy(p1_hbm.at[pl.ds(nb, NPT)], p1buf)

    def comb_body(i, c):
        sl = pl.ds(i * L, L)
        cur = p0buf[sl] + p1buf[sl] + bvec[...]
        p0buf[sl] = cur * cvec[...]
        p1buf[sl] = 1.0 - cur
        zbuf[sl] = jnp.zeros((L,), _f32)
        return c
    lax.fori_loop(0, NPT // L, comb_body, 0)

    pltpu.sync_copy(p0buf, cur_s.at[pl.ds(nb, NPT)])
    pltpu.sync_copy(zbuf, acc_s.at[pl.ds(nb, NPT)])

    @pl.when(cid == 0)
    def _():
        pltpu.sync_copy(surv_hbm.at[pl.ds(nb, NPT)], sbuf)
        def sbody(i, c):
            sl = pl.ds(i * L, L)
            sbuf[sl] = sbuf[sl] * p1buf[sl]
            return c
        lax.fori_loop(0, NPT // L, sbody, 0)
        pltpu.sync_copy(sbuf, surv_out.at[pl.ds(nb, NPT)])

    plsc.subcore_barrier()
    _edge_pass(src_hbm, dst_hbm, ep_hbm, cur_s, acc_s,
               (s0, s1, s2), (d0, d1, d2), (pr0, pr1, pr2), (v0, v1),
               (ls0, ls1, ls2), gsem, (ss0, ss1), cid, sid)
    plsc.subcore_barrier()
    _write_partial(acc_s, p0_out, p1_out, cid, nb)


def _body_fin(p0_hbm, p1_hbm, surv_hbm, bias_hbm, out_hbm,
              p0buf, p1buf, sbuf, bvec):
    cid = lax.axis_index("c")
    sid = lax.axis_index("s")
    nb = (cid * NS + sid) * NPW

    pltpu.sync_copy(bias_hbm, bvec)
    pltpu.sync_copy(p0_hbm.at[pl.ds(nb, NPW)], p0buf)
    pltpu.sync_copy(p1_hbm.at[pl.ds(nb, NPW)], p1buf)
    pltpu.sync_copy(surv_hbm.at[pl.ds(nb, NPW)], sbuf)

    def fin_body(i, c):
        sl = pl.ds(i * L, L)
        cur = p0buf[sl] + p1buf[sl] + bvec[...]
        v = 1.0 - sbuf[sl] * (1.0 - cur)
        p0buf[sl] = jnp.minimum(jnp.maximum(v, 0.0), 1.0)
        return c
    lax.fori_loop(0, NPW // L, fin_body, 0)

    pltpu.sync_copy(p0buf, out_hbm.at[pl.ds(nb, NPW)])


def kernel(x, edge_index, edge_probs, time_decay, node_bias, edge_weight):
    x_pad = jnp.pad(x[:, 0], (0, NPAD - N))
    ei = edge_index.astype(jnp.int32)
    ep = edge_probs.astype(_f32)
    coefs = edge_weight.astype(_f32) * jnp.exp(-(time_decay.astype(_f32) ** 2))
    coefs16 = jnp.broadcast_to(coefs[:, None], (NSTEPS, L))
    bias16 = jnp.full((L,), node_bias, _f32)

    mesh = plsc.VectorSubcoreMesh(core_axis_name="c", subcore_axis_name="s",
                                  num_cores=NC, num_subcores=NS)
    node_arr = jax.ShapeDtypeStruct((NPAD,), _f32)

    edge_scratch = (
        [pltpu.VMEM((CHUNK,), jnp.int32)] * 6
        + [pltpu.VMEM((CHUNK,), _f32)] * 5
        + [pltpu.SemaphoreType.DMA] * 6
    )
    spmem_scratch = [
        pltpu.VMEM_SHARED((NPAD,), _f32),
        pltpu.VMEM_SHARED((NPAD,), _f32),
    ]

    first = pl.kernel(
        _body_first,
        out_type=(node_arr, node_arr, node_arr),
        mesh=mesh,
        scratch_types=spmem_scratch + [
            pltpu.VMEM((NPT,), _f32),
            pltpu.VMEM((NPT,), _f32),
            pltpu.VMEM((NPT,), _f32),
        ] + edge_scratch + [
            pltpu.VMEM((L,), _f32),
        ],
    )
    mid = pl.kernel(
        _body_mid,
        out_type=(node_arr, node_arr, node_arr),
        mesh=mesh,
        scratch_types=spmem_scratch + [
            pltpu.VMEM((NPT,), _f32),
            pltpu.VMEM((NPT,), _f32),
            pltpu.VMEM((NPT,), _f32),
            pltpu.VMEM((NPT,), _f32),
        ] + edge_scratch + [
            pltpu.VMEM((L,), _f32),
            pltpu.VMEM((L,), _f32),
        ],
    )
    fin = pl.kernel(
        _body_fin,
        out_type=node_arr,
        mesh=mesh,
        scratch_types=[
            pltpu.VMEM((NPW,), _f32),
            pltpu.VMEM((NPW,), _f32),
            pltpu.VMEM((NPW,), _f32),
            pltpu.VMEM((L,), _f32),
        ],
    )

    p0, p1, surv = first(x_pad, ei[0], ei[1], ep, coefs16[0])
    for k in range(1, NSTEPS):
        p0, p1, surv = mid(p0, p1, surv, ei[0], ei[1], ep,
                           coefs16[k], bias16)
    out = fin(p0, p1, surv, bias16)
    return out[:N, None]


# speedup vs baseline: 394.2185x; 1.1914x over previous
import jax
import jax.numpy as jnp
from jax import lax
from jax.experimental import pallas as pl
from jax.experimental.pallas import tpu as pltpu
from jax.experimental.pallas import tpu_sc as plsc

NSTEPS = 8
N = 100000
E = 6400000
NC, NS, L = 2, 16, 16
NPAD = 102400
NPT = NPAD // NS
NPW = NPAD // (NC * NS)
EPSC = E // NC
EPT = EPSC // NS
CHUNK = 8000
NCHUNKS = EPT // CHUNK

_f32 = jnp.float32


def _edge_pass(src_hbm, dst_hbm, ep_hbm, cur_s, acc_s,
               srcs, dsts, probs, vals, lsems, gsems, ssems, cid, sid):
    ldesc, sdesc = {}, {}

    def issue_loads(c):
        b = c % 3
        base = cid * EPSC + sid * EPT + c * CHUNK
        ldesc[c] = (
            pltpu.async_copy(src_hbm.at[pl.ds(base, CHUNK)], srcs[b], lsems[b]),
            pltpu.async_copy(dst_hbm.at[pl.ds(base, CHUNK)], dsts[b], lsems[b]),
            pltpu.async_copy(ep_hbm.at[pl.ds(base, CHUNK)], probs[b], lsems[b]),
        )

    issue_loads(0)
    issue_loads(1)
    for c in range(NCHUNKS):
        b, vb = c % 3, c % 2
        for d in ldesc.pop(c):
            d.wait()
        if c >= 2:
            sdesc.pop(c - 2).wait()
        if c + 1 < NCHUNKS and c + 1 not in ldesc:
            issue_loads(c + 1)
        Q = CHUNK // 4
        gd = {}
        def issue_gather(q, b=b, vb=vb):
            sl = pl.ds(q * Q, Q)
            gd[q] = pltpu.async_copy(cur_s.at[srcs[b].at[sl]],
                                     vals[vb].at[sl], gsems[q % 2])
        issue_gather(0)
        issue_gather(1)
        for q in range(4):
            gd.pop(q).wait()
            if q + 2 < 4:
                issue_gather(q + 2)
            def mul_body(i, c2, q=q, vb=vb, b=b):
                sl = pl.ds(q * Q + i * L, L)
                vals[vb][sl] = vals[vb][sl] * probs[b][sl]
                return c2
            lax.fori_loop(0, Q // L, mul_body, 0)
        sdesc[c] = pltpu.async_copy(vals[vb], acc_s.at[dsts[b]], ssems[vb],
                                    add=True)
    sdesc.pop(NCHUNKS - 2).wait()
    sdesc.pop(NCHUNKS - 1).wait()


def _write_partial(acc_s, p0_out, p1_out, cid, nb):
    sl = pl.ds(nb, NPT)
    @pl.when(cid == 0)
    def _():
        pltpu.sync_copy(acc_s.at[sl], p0_out.at[sl])
    @pl.when(cid == 1)
    def _():
        pltpu.sync_copy(acc_s.at[sl], p1_out.at[sl])


def _body_first(x_hbm, src_hbm, dst_hbm, ep_hbm, coef_hbm,
                p0_out, p1_out, surv_out,
                cur_s, acc_s, xbuf, abuf, zbuf,
                s0, s1, s2, d0, d1, d2, pr0, pr1, pr2, v0, v1,
                ls0, ls1, ls2, gs0, gs1, ss0, ss1,
                cvec):
    cid = lax.axis_index("c")
    sid = lax.axis_index("s")
    nb = sid * NPT

    pltpu.sync_copy(coef_hbm, cvec)
    pltpu.sync_copy(x_hbm.at[pl.ds(nb, NPT)], xbuf)

    def init_body(i, c):
        sl = pl.ds(i * L, L)
        abuf[sl] = cvec[...] * xbuf[sl]
        zbuf[sl] = jnp.zeros((L,), _f32)
        return c
    lax.fori_loop(0, NPT // L, init_body, 0)

    pltpu.sync_copy(abuf, cur_s.at[pl.ds(nb, NPT)])
    pltpu.sync_copy(zbuf, acc_s.at[pl.ds(nb, NPT)])

    @pl.when(cid == 0)
    def _():
        def sbody(i, c):
            sl = pl.ds(i * L, L)
            xbuf[sl] = 1.0 - xbuf[sl]
            return c
        lax.fori_loop(0, NPT // L, sbody, 0)
        pltpu.sync_copy(xbuf, surv_out.at[pl.ds(nb, NPT)])

    plsc.subcore_barrier()
    _edge_pass(src_hbm, dst_hbm, ep_hbm, cur_s, acc_s,
               (s0, s1, s2), (d0, d1, d2), (pr0, pr1, pr2), (v0, v1),
               (ls0, ls1, ls2), (gs0, gs1), (ss0, ss1), cid, sid)
    plsc.subcore_barrier()
    _write_partial(acc_s, p0_out, p1_out, cid, nb)


def _body_mid(p0_hbm, p1_hbm, surv_hbm, src_hbm, dst_hbm, ep_hbm,
              coef_hbm, bias_hbm,
              p0_out, p1_out, surv_out,
              cur_s, acc_s, p0buf, p1buf, sbuf, zbuf,
              s0, s1, s2, d0, d1, d2, pr0, pr1, pr2, v0, v1,
              ls0, ls1, ls2, gs0, gs1, ss0, ss1,
              cvec, bvec):
    cid = lax.axis_index("c")
    sid = lax.axis_index("s")
    nb = sid * NPT

    pltpu.sync_copy(coef_hbm, cvec)
    pltpu.sync_copy(bias_hbm, bvec)
    pltpu.sync_copy(p0_hbm.at[pl.ds(nb, NPT)], p0buf)
    pltpu.sync_copy(p1_hbm.at[pl.ds(nb, NPT)], p1buf)

    def comb_body(i, c):
        sl = pl.ds(i * L, L)
        cur = p0buf[sl] + p1buf[sl] + bvec[...]
        p0buf[sl] = cur * cvec[...]
        p1buf[sl] = 1.0 - cur
        zbuf[sl] = jnp.zeros((L,), _f32)
        return c
    lax.fori_loop(0, NPT // L, comb_body, 0)

    pltpu.sync_copy(p0buf, cur_s.at[pl.ds(nb, NPT)])
    pltpu.sync_copy(zbuf, acc_s.at[pl.ds(nb, NPT)])

    @pl.when(cid == 0)
    def _():
        pltpu.sync_copy(surv_hbm.at[pl.ds(nb, NPT)], sbuf)
        def sbody(i, c):
            sl = pl.ds(i * L, L)
            sbuf[sl] = sbuf[sl] * p1buf[sl]
            return c
        lax.fori_loop(0, NPT // L, sbody, 0)
        pltpu.sync_copy(sbuf, surv_out.at[pl.ds(nb, NPT)])

    plsc.subcore_barrier()
    _edge_pass(src_hbm, dst_hbm, ep_hbm, cur_s, acc_s,
               (s0, s1, s2), (d0, d1, d2), (pr0, pr1, pr2), (v0, v1),
               (ls0, ls1, ls2), (gs0, gs1), (ss0, ss1), cid, sid)
    plsc.subcore_barrier()
    _write_partial(acc_s, p0_out, p1_out, cid, nb)


def _body_fin(p0_hbm, p1_hbm, surv_hbm, bias_hbm, out_hbm,
              p0buf, p1buf, sbuf, bvec):
    cid = lax.axis_index("c")
    sid = lax.axis_index("s")
    nb = (cid * NS + sid) * NPW

    pltpu.sync_copy(bias_hbm, bvec)
    pltpu.sync_copy(p0_hbm.at[pl.ds(nb, NPW)], p0buf)
    pltpu.sync_copy(p1_hbm.at[pl.ds(nb, NPW)], p1buf)
    pltpu.sync_copy(surv_hbm.at[pl.ds(nb, NPW)], sbuf)

    def fin_body(i, c):
        sl = pl.ds(i * L, L)
        cur = p0buf[sl] + p1buf[sl] + bvec[...]
        v = 1.0 - sbuf[sl] * (1.0 - cur)
        p0buf[sl] = jnp.minimum(jnp.maximum(v, 0.0), 1.0)
        return c
    lax.fori_loop(0, NPW // L, fin_body, 0)

    pltpu.sync_copy(p0buf, out_hbm.at[pl.ds(nb, NPW)])


def kernel(x, edge_index, edge_probs, time_decay, node_bias, edge_weight):
    x_pad = jnp.pad(x[:, 0], (0, NPAD - N))
    ei = edge_index.astype(jnp.int32)
    ep = edge_probs.astype(_f32)
    coefs = edge_weight.astype(_f32) * jnp.exp(-(time_decay.astype(_f32) ** 2))
    coefs16 = jnp.broadcast_to(coefs[:, None], (NSTEPS, L))
    bias16 = jnp.full((L,), node_bias, _f32)

    mesh = plsc.VectorSubcoreMesh(core_axis_name="c", subcore_axis_name="s",
                                  num_cores=NC, num_subcores=NS)
    node_arr = jax.ShapeDtypeStruct((NPAD,), _f32)

    edge_scratch = (
        [pltpu.VMEM((CHUNK,), jnp.int32)] * 6
        + [pltpu.VMEM((CHUNK,), _f32)] * 5
        + [pltpu.SemaphoreType.DMA] * 7
    )
    spmem_scratch = [
        pltpu.VMEM_SHARED((NPAD,), _f32),
        pltpu.VMEM_SHARED((NPAD,), _f32),
    ]

    first = pl.kernel(
        _body_first,
        out_type=(node_arr, node_arr, node_arr),
        mesh=mesh,
        scratch_types=spmem_scratch + [
            pltpu.VMEM((NPT,), _f32),
            pltpu.VMEM((NPT,), _f32),
            pltpu.VMEM((NPT,), _f32),
        ] + edge_scratch + [
            pltpu.VMEM((L,), _f32),
        ],
    )
    mid = pl.kernel(
        _body_mid,
        out_type=(node_arr, node_arr, node_arr),
        mesh=mesh,
        scratch_types=spmem_scratch + [
            pltpu.VMEM((NPT,), _f32),
            pltpu.VMEM((NPT,), _f32),
            pltpu.VMEM((NPT,), _f32),
            pltpu.VMEM((NPT,), _f32),
        ] + edge_scratch + [
            pltpu.VMEM((L,), _f32),
            pltpu.VMEM((L,), _f32),
        ],
    )
    fin = pl.kernel(
        _body_fin,
        out_type=node_arr,
        mesh=mesh,
        scratch_types=[
            pltpu.VMEM((NPW,), _f32),
            pltpu.VMEM((NPW,), _f32),
            pltpu.VMEM((NPW,), _f32),
            pltpu.VMEM((L,), _f32),
        ],
    )

    p0, p1, surv = first(x_pad, ei[0], ei[1], ep, coefs16[0])
    for k in range(1, NSTEPS):
        p0, p1, surv = mid(p0, p1, surv, ei[0], ei[1], ep,
                           coefs16[k], bias16)
    out = fin(p0, p1, surv, bias16)
    return out[:N, None]
